# Initial kernel scaffold; baseline (speedup 1.0000x reference)
import jax
import jax.numpy as jnp
from jax.experimental import pallas as pl


def kernel(cat_feats, num_feats, tables, W_num, b_num, W_final, b_final):
    raise NotImplementedError("write your pallas kernel here")


# speedup vs baseline: 5.0748x; 5.0748x over previous
import functools

import jax
import jax.numpy as jnp
from jax import lax
from jax.experimental import pallas as pl
from jax.experimental.pallas import tpu as pltpu
from jax.experimental.pallas import tpu_sc as plsc


def _sc_gather(table_flat, idx2d, n_rows, embed, n_workers, chunk):
    per_w = n_rows // n_workers
    iters = per_w // chunk
    ki = chunk // 128

    mesh = plsc.VectorSubcoreMesh(core_axis_name="c", subcore_axis_name="s")

    @functools.partial(
        pl.kernel,
        out_type=jax.ShapeDtypeStruct((n_rows, embed), jnp.float32),
        mesh=mesh,
        scratch_types=[
            pltpu.VMEM((ki, 128), jnp.int32),
            pltpu.VMEM((chunk, embed), jnp.float32),
            pltpu.SemaphoreType.DMA,
        ],
        compiler_params=pltpu.CompilerParams(use_tc_tiling_on_sc=False),
    )
    def k(idx_hbm, tab_hbm, out_hbm, idx_v, rows_v, sem):
        n_cores = 2
        wid = lax.axis_index("s") * n_cores + lax.axis_index("c")
        base128 = wid * (per_w // 128)

        def body(i, carry):
            blk = base128 + i * ki
            pltpu.sync_copy(idx_hbm.at[pl.ds(blk, ki)], idx_v)
            copies = [
                pltpu.async_copy(
                    tab_hbm.at[idx_v.at[j]],
                    rows_v.at[pl.ds(j * 128, 128)],
                    sem,
                )
                for j in range(ki)
            ]
            for c in copies:
                c.wait()
            pltpu.sync_copy(rows_v, out_hbm.at[pl.ds(blk * 128, chunk)])
            return carry

        lax.fori_loop(0, iters, body, 0)

    return k(idx2d, table_flat)


def _tc_tail(g, num2d, w_num, b_num, w_final, b_final, block_t):
    t, cat_d = g.shape
    num_dim = num2d.shape[1]
    embed = w_num.shape[1]
    d_model = w_final.shape[1]
    grid = (t // block_t,)

    def body(g_ref, n_ref, wf_ref, wn_ref, bn_ref, bf_ref, o_ref):
        wc = wf_ref[:cat_d, :]
        wt = wf_ref[cat_d:, :]
        nump = (
            jnp.dot(n_ref[...], wn_ref[...], preferred_element_type=jnp.float32)
            + bn_ref[...]
        )
        o_ref[...] = (
            jnp.dot(g_ref[...], wc, preferred_element_type=jnp.float32)
            + jnp.dot(nump, wt, preferred_element_type=jnp.float32)
            + bf_ref[...]
        )

    return pl.pallas_call(
        body,
        grid=grid,
        in_specs=[
            pl.BlockSpec((block_t, cat_d), lambda i: (i, 0)),
            pl.BlockSpec((block_t, num_dim), lambda i: (i, 0)),
            pl.BlockSpec((cat_d + embed, d_model), lambda i: (0, 0)),
            pl.BlockSpec((num_dim, embed), lambda i: (0, 0)),
            pl.BlockSpec((1, embed), lambda i: (0, 0)),
            pl.BlockSpec((1, d_model), lambda i: (0, 0)),
        ],
        out_specs=pl.BlockSpec((block_t, d_model), lambda i: (i, 0)),
        out_shape=jax.ShapeDtypeStruct((t, d_model), jnp.float32),
    )(g, num2d, w_final, w_num, b_num.reshape(1, embed), b_final.reshape(1, d_model))


def kernel(cat_feats, num_feats, tables, W_num, b_num, W_final, b_final):
    b, l, n_cat = cat_feats.shape
    _, vocab, embed = tables.shape
    num_dim = num_feats.shape[-1]
    d_model = W_final.shape[1]
    t = b * l
    n_rows = t * n_cat

    gidx = cat_feats.astype(jnp.int32) + (
        jnp.arange(n_cat, dtype=jnp.int32) * vocab
    )
    idx2d = gidx.reshape(n_rows // 128, 128)
    table_flat = tables.reshape(n_cat * vocab, embed)

    gathered = _sc_gather(
        table_flat, idx2d, n_rows, embed, n_workers=32, chunk=1024
    )
    g = gathered.reshape(t, n_cat * embed)

    out = _tc_tail(
        g, num_feats.reshape(t, num_dim), W_num, b_num, W_final, b_final,
        block_t=2048,
    )
    return out.reshape(b, l, d_model)
